# Initial kernel scaffold; baseline (speedup 1.0000x reference)
import jax
import jax.numpy as jnp
from jax.experimental import pallas as pl


def kernel(token_ids, word_embeddings, positional_embeddings):
    raise NotImplementedError("write your pallas kernel here")


# speedup vs baseline: 2.0944x; 2.0944x over previous
import functools

import jax
import jax.numpy as jnp
from jax import lax
from jax.experimental import pallas as pl
from jax.experimental.pallas import tpu as pltpu
from jax.experimental.pallas import tpu_sc as plsc

_LANES = 16
_CHUNK = 128


@functools.cache
def _build(B, T, V, D):
    info = plsc.get_sparse_core_info()
    NC, NS = info.num_cores, info.num_subcores
    NW = NC * NS
    FLAT = B * T
    assert FLAT % (NW * _CHUNK) == 0 and D % _LANES == 0
    per_w = FLAT // NW
    assert per_w % T == 0 and T % _CHUNK == 0
    n_chunks = per_w // _CHUNK
    mesh = plsc.VectorSubcoreMesh(core_axis_name="c", subcore_axis_name="s")

    @functools.partial(
        pl.kernel,
        mesh=mesh,
        out_type=jax.ShapeDtypeStruct((FLAT, D), jnp.float32),
        scratch_types=[
            pltpu.VMEM((n_chunks, _CHUNK), jnp.int32),
            pltpu.VMEM((T, D), jnp.float32),
            pltpu.VMEM((_CHUNK, D), jnp.float32),
            pltpu.SemaphoreType.DMA,
        ],
    )
    def k(tok_hbm, table_hbm, pos_hbm, out_hbm, idx_v, pos_v, rows_v, sem):
        wid = lax.axis_index("s") * NC + lax.axis_index("c")
        pltpu.sync_copy(tok_hbm.at[wid], idx_v)
        pltpu.sync_copy(pos_hbm, pos_v)
        base = wid * per_w

        def chunk_body(c, carry):
            pltpu.async_copy(table_hbm.at[idx_v.at[c]], rows_v, sem).wait()
            p0 = lax.rem(c * _CHUNK, T)

            def add_row(r, carry2):
                for j in range(D // _LANES):
                    sl = pl.ds(j * _LANES, _LANES)
                    rows_v[r, sl] = rows_v[r, sl] + pos_v[p0 + r, sl]
                return carry2

            lax.fori_loop(0, _CHUNK, add_row, 0, unroll=2)
            pltpu.sync_copy(rows_v, out_hbm.at[pl.ds(base + c * _CHUNK, _CHUNK)])
            return carry

        lax.fori_loop(0, n_chunks, chunk_body, 0)

    return k


def kernel(token_ids, word_embeddings, positional_embeddings):
    B, T = token_ids.shape
    V, D = word_embeddings.shape
    k = _build(B, T, V, D)
    info = plsc.get_sparse_core_info()
    NW = info.num_cores * info.num_subcores
    tok = token_ids.astype(jnp.int32).reshape(NW, (B * T) // (NW * _CHUNK), _CHUNK)
    out = k(tok, word_embeddings, positional_embeddings)
    return out.reshape(B, T, D)


# speedup vs baseline: 2.5349x; 1.2103x over previous
import functools

import jax
import jax.numpy as jnp
from jax import lax
from jax.experimental import pallas as pl
from jax.experimental.pallas import tpu as pltpu
from jax.experimental.pallas import tpu_sc as plsc

_LANES = 16
_CHUNK = 128


@functools.cache
def _build(B, T, V, D):
    info = plsc.get_sparse_core_info()
    NC, NS = info.num_cores, info.num_subcores
    NW = NC * NS
    FLAT = B * T
    assert FLAT % (NW * _CHUNK) == 0 and D % _LANES == 0
    per_w = FLAT // NW
    assert per_w % T == 0 and T % _CHUNK == 0
    n_chunks = per_w // _CHUNK
    mesh = plsc.VectorSubcoreMesh(core_axis_name="c", subcore_axis_name="s")

    @functools.partial(
        pl.kernel,
        mesh=mesh,
        out_type=jax.ShapeDtypeStruct((FLAT, D), jnp.float32),
        scratch_types=[
            pltpu.VMEM((n_chunks, _CHUNK), jnp.int32),
            pltpu.VMEM((T, D), jnp.float32),
            pltpu.VMEM((_CHUNK, D), jnp.float32),
            pltpu.VMEM((_CHUNK, D), jnp.float32),
            pltpu.SemaphoreType.DMA,
            pltpu.SemaphoreType.DMA,
            pltpu.SemaphoreType.DMA,
            pltpu.SemaphoreType.DMA,
        ],
    )
    def k(tok_hbm, table_hbm, pos_hbm, out_hbm, idx_v, pos_v,
          rows0, rows1, g0, g1, s0, s1):
        rows = (rows0, rows1)
        gsem = (g0, g1)
        ssem = (s0, s1)
        wid = lax.axis_index("s") * NC + lax.axis_index("c")
        pltpu.sync_copy(tok_hbm.at[wid], idx_v)
        pltpu.sync_copy(pos_hbm, pos_v)
        base = wid * per_w

        def add_pos(rows_v, c):
            p0 = lax.rem(c * _CHUNK, T)

            def add_row(r, carry2):
                for j in range(D // _LANES):
                    sl = pl.ds(j * _LANES, _LANES)
                    rows_v[r, sl] = rows_v[r, sl] + pos_v[p0 + r, sl]
                return carry2

            lax.fori_loop(0, _CHUNK, add_row, 0, unroll=2)

        pltpu.async_copy(table_hbm.at[idx_v.at[0]], rows0, g0)

        def loop_body(g, carry):
            for b in (0, 1):
                c = 2 * g + b
                pltpu.make_async_copy(
                    table_hbm.at[idx_v.at[c]], rows[b], gsem[b]).wait()
                if b == 0:
                    @pl.when(g > 0)
                    def _():
                        pltpu.make_async_copy(
                            rows[1], out_hbm.at[pl.ds(0, _CHUNK)], ssem[1]).wait()
                    pltpu.async_copy(
                        table_hbm.at[idx_v.at[c + 1]], rows[1], gsem[1])
                else:
                    pltpu.make_async_copy(
                        rows[0], out_hbm.at[pl.ds(0, _CHUNK)], ssem[0]).wait()

                    @pl.when(g < n_chunks // 2 - 1)
                    def _():
                        pltpu.async_copy(
                            table_hbm.at[idx_v.at[c + 1]], rows[0], gsem[0])
                add_pos(rows[b], c)
                pltpu.async_copy(
                    rows[b], out_hbm.at[pl.ds(base + c * _CHUNK, _CHUNK)], ssem[b])
            return carry

        lax.fori_loop(0, n_chunks // 2, loop_body, 0)
        pltpu.make_async_copy(rows1, out_hbm.at[pl.ds(0, _CHUNK)], s1).wait()

    return k


def kernel(token_ids, word_embeddings, positional_embeddings):
    B, T = token_ids.shape
    V, D = word_embeddings.shape
    k = _build(B, T, V, D)
    info = plsc.get_sparse_core_info()
    NW = info.num_cores * info.num_subcores
    tok = token_ids.astype(jnp.int32).reshape(NW, (B * T) // (NW * _CHUNK), _CHUNK)
    out = k(tok, word_embeddings, positional_embeddings)
    return out.reshape(B, T, D)


# speedup vs baseline: 7.1690x; 2.8282x over previous
import functools

import jax
import jax.numpy as jnp
from jax import lax
from jax.experimental import pallas as pl
from jax.experimental.pallas import tpu as pltpu
from jax.experimental.pallas import tpu_sc as plsc

_LANES = 16
_CHUNK = 128


@functools.cache
def _build(B, T, V, D):
    info = plsc.get_sparse_core_info()
    NC, NS = info.num_cores, info.num_subcores
    NW = NC * NS
    FLAT = B * T
    assert FLAT % (NW * _CHUNK) == 0 and D % _LANES == 0
    per_w = FLAT // NW
    assert per_w % T == 0 and T % _CHUNK == 0
    n_chunks = per_w // _CHUNK
    mesh = plsc.VectorSubcoreMesh(core_axis_name="c", subcore_axis_name="s")

    @functools.partial(
        pl.kernel,
        mesh=mesh,
        out_type=jax.ShapeDtypeStruct((FLAT, D), jnp.float32),
        scratch_types=[
            pltpu.VMEM((n_chunks, _CHUNK), jnp.int32),
            pltpu.VMEM((T, D), jnp.float32),
            pltpu.VMEM((_CHUNK, D), jnp.float32),
            pltpu.VMEM((_CHUNK, D), jnp.float32),
            pltpu.SemaphoreType.DMA,
            pltpu.SemaphoreType.DMA,
            pltpu.SemaphoreType.DMA,
            pltpu.SemaphoreType.DMA,
        ],
    )
    def k(tok_hbm, table_hbm, pos_hbm, out_hbm, idx_v, pos_v,
          rows0, rows1, g0, g1, s0, s1):
        rows = (rows0, rows1)
        gsem = (g0, g1)
        ssem = (s0, s1)
        wid = lax.axis_index("s") * NC + lax.axis_index("c")
        pltpu.sync_copy(tok_hbm.at[wid], idx_v)
        pltpu.sync_copy(pos_hbm, pos_v)
        base = wid * per_w

        def add_pos(rows_v, c):
            p0 = lax.rem(c * _CHUNK, T)

            @plsc.parallel_loop(0, _CHUNK, 1, unroll=4)
            def add_row(r):
                for j in range(D // _LANES):
                    sl = pl.ds(j * _LANES, _LANES)
                    rows_v[r, sl] = rows_v[r, sl] + pos_v[p0 + r, sl]

        pltpu.async_copy(table_hbm.at[idx_v.at[0]], rows0, g0)

        def loop_body(g, carry):
            for b in (0, 1):
                c = 2 * g + b
                pltpu.make_async_copy(
                    table_hbm.at[idx_v.at[c]], rows[b], gsem[b]).wait()
                if b == 0:
                    @pl.when(g > 0)
                    def _():
                        pltpu.make_async_copy(
                            rows[1], out_hbm.at[pl.ds(0, _CHUNK)], ssem[1]).wait()
                    pltpu.async_copy(
                        table_hbm.at[idx_v.at[c + 1]], rows[1], gsem[1])
                else:
                    pltpu.make_async_copy(
                        rows[0], out_hbm.at[pl.ds(0, _CHUNK)], ssem[0]).wait()

                    @pl.when(g < n_chunks // 2 - 1)
                    def _():
                        pltpu.async_copy(
                            table_hbm.at[idx_v.at[c + 1]], rows[0], gsem[0])
                add_pos(rows[b], c)
                pltpu.async_copy(
                    rows[b], out_hbm.at[pl.ds(base + c * _CHUNK, _CHUNK)], ssem[b])
            return carry

        lax.fori_loop(0, n_chunks // 2, loop_body, 0)
        pltpu.make_async_copy(rows1, out_hbm.at[pl.ds(0, _CHUNK)], s1).wait()

    return k


def kernel(token_ids, word_embeddings, positional_embeddings):
    B, T = token_ids.shape
    V, D = word_embeddings.shape
    k = _build(B, T, V, D)
    info = plsc.get_sparse_core_info()
    NW = info.num_cores * info.num_subcores
    tok = token_ids.astype(jnp.int32).reshape(NW, (B * T) // (NW * _CHUNK), _CHUNK)
    out = k(tok, word_embeddings, positional_embeddings)
    return out.reshape(B, T, D)
